# Initial kernel scaffold; baseline (speedup 1.0000x reference)
import jax
import jax.numpy as jnp
from jax.experimental import pallas as pl


def kernel(output, x_A, y_A, x_B, y_B, ordinal_relation):
    raise NotImplementedError("write your pallas kernel here")


# speedup vs baseline: 4.5076x; 4.5076x over previous
import functools

import jax
import jax.numpy as jnp
from jax import lax
from jax.experimental import pallas as pl
from jax.experimental.pallas import tpu as pltpu
from jax.experimental.pallas import tpu_sc as plsc

B, H, W, P = 16, 512, 512, 20000
PPAD = 20096
NROWS = 2 * B
LANES = 16


def _gather_body(x_hbm, y_hbm, depth_hbm, out_hbm, xv, yv, idxv, valsv, sem):
    wid = lax.axis_index("s") * 2 + lax.axis_index("c")
    img = lax.rem(wid, B)
    base = img * (H * W)
    pltpu.sync_copy(x_hbm.at[wid], xv)
    pltpu.sync_copy(y_hbm.at[wid], yv)

    def body(j, carry):
        sl = pl.ds(j * LANES, LANES)
        xi = xv[sl]
        yi = yv[sl]
        idxv[sl] = (xi - 10) * W + (yi - 10) + base
        return carry

    lax.fori_loop(0, PPAD // LANES, body, 0)
    pltpu.async_copy(depth_hbm.at[idxv], valsv, sem).wait()
    pltpu.sync_copy(valsv, out_hbm.at[wid])


@functools.partial(jax.jit, static_argnums=())
def _gather(x, y, depth):
    mesh = plsc.VectorSubcoreMesh(core_axis_name="c", subcore_axis_name="s")
    run = pl.kernel(
        _gather_body,
        mesh=mesh,
        out_type=jax.ShapeDtypeStruct((NROWS, PPAD), jnp.float32),
        scratch_types=[
            pltpu.VMEM((PPAD,), jnp.int32),
            pltpu.VMEM((PPAD,), jnp.int32),
            pltpu.VMEM((PPAD,), jnp.int32),
            pltpu.VMEM((PPAD,), jnp.float32),
            pltpu.SemaphoreType.DMA,
        ],
    )
    return run(x, y, depth)


def _loss_body(vals_ref, t_ref, out_ref):
    z_a = vals_ref[0:B, :]
    z_b = vals_ref[B:NROWS, :]
    t = t_ref[...]
    pred = z_a - z_b
    mask = t != 0.0
    log_terms = jnp.where(mask, jnp.log1p(jnp.exp(-t * pred)), 0.0)
    sq_terms = jnp.where(mask, pred * pred, 0.0)
    cnt = jnp.sum(mask.astype(jnp.float32), axis=1)
    per_image = (jnp.sum(log_terms, axis=1) + jnp.sum(sq_terms, axis=1)) / cnt
    out_ref[...] = (jnp.sum(per_image) / jnp.float32(B)).reshape(1, 1)


def kernel(output, x_A, y_A, x_B, y_B, ordinal_relation):
    pad = PPAD - P
    x = jnp.pad(jnp.concatenate([x_A, x_B], axis=0).astype(jnp.int32),
                ((0, 0), (0, pad)), constant_values=10)
    y = jnp.pad(jnp.concatenate([y_A, y_B], axis=0).astype(jnp.int32),
                ((0, 0), (0, pad)), constant_values=10)
    t = jnp.pad(ordinal_relation.astype(jnp.float32), ((0, 0), (0, pad)))
    depth = output.reshape(B * H * W)
    vals = _gather(x, y, depth)
    loss = pl.pallas_call(
        _loss_body,
        out_shape=jax.ShapeDtypeStruct((1, 1), jnp.float32),
    )(vals, t)
    return loss[0, 0]


# speedup vs baseline: 4.7986x; 1.0646x over previous
import jax
import jax.numpy as jnp
from jax import lax
from jax.experimental import pallas as pl
from jax.experimental.pallas import tpu as pltpu
from jax.experimental.pallas import tpu_sc as plsc

B, H, W, P = 16, 512, 512, 20000
NROWS = 2 * B
PPAD = 20096
LANES = 16
NCH = 10
CH = P // NCH
IT = CH // LANES


def _gather_body(xa, ya, xb, yb, depth, out, xv, yv, idxv, valsv, sem):
    wid = lax.axis_index("s") * 2 + lax.axis_index("c")
    img = lax.rem(wid, B)
    base = img * (H * W)

    row = pl.ds(img * P, P)

    @pl.when(wid < B)
    def _():
        pltpu.sync_copy(xa.at[row], xv)
        pltpu.sync_copy(ya.at[row], yv)

    @pl.when(wid >= B)
    def _():
        pltpu.sync_copy(xb.at[row], xv)
        pltpu.sync_copy(yb.at[row], yv)

    gathers = []
    for k in range(NCH):
        def body(j, carry, _k=k):
            sl = pl.ds(_k * CH + j * LANES, LANES)
            idxv[sl] = (xv[sl] - 10) * W + (yv[sl] - 10) + base
            return carry

        lax.fori_loop(0, IT, body, 0)
        gathers.append(pltpu.async_copy(
            depth.at[idxv.at[pl.ds(k * CH, CH)]],
            valsv.at[pl.ds(k * CH, CH)], sem))
    for d in gathers:
        d.wait()
    pltpu.sync_copy(valsv, out.at[wid])


def _gather(xa, ya, xb, yb, depth):
    mesh = plsc.VectorSubcoreMesh(core_axis_name="c", subcore_axis_name="s")
    run = pl.kernel(
        _gather_body,
        mesh=mesh,
        out_type=jax.ShapeDtypeStruct((NROWS, PPAD), jnp.float32),
        scratch_types=[
            pltpu.VMEM((P,), jnp.int32),
            pltpu.VMEM((P,), jnp.int32),
            pltpu.VMEM((P,), jnp.int32),
            pltpu.VMEM((PPAD,), jnp.float32),
            pltpu.SemaphoreType.DMA,
        ],
    )
    return run(xa, ya, xb, yb, depth)


def _loss_body(vals_ref, t_ref, out_ref):
    z_a = vals_ref[0:B, 0:P]
    z_b = vals_ref[B:NROWS, 0:P]
    t = t_ref[...]
    pred = z_a - z_b
    mask = t != 0.0
    log_terms = jnp.where(mask, jnp.log1p(jnp.exp(-t * pred)), 0.0)
    sq_terms = jnp.where(mask, pred * pred, 0.0)
    cnt = jnp.sum(mask.astype(jnp.float32), axis=1)
    per_image = (jnp.sum(log_terms, axis=1) + jnp.sum(sq_terms, axis=1)) / cnt
    out_ref[...] = (jnp.sum(per_image) / jnp.float32(B)).reshape(1, 1)


def kernel(output, x_A, y_A, x_B, y_B, ordinal_relation):
    depth = output.reshape(B * H * W)
    vals = _gather(x_A.astype(jnp.int32).reshape(B * P),
                   y_A.astype(jnp.int32).reshape(B * P),
                   x_B.astype(jnp.int32).reshape(B * P),
                   y_B.astype(jnp.int32).reshape(B * P),
                   depth)
    loss = pl.pallas_call(
        _loss_body,
        out_shape=jax.ShapeDtypeStruct((1, 1), jnp.float32),
    )(vals, ordinal_relation.astype(jnp.float32))
    return loss[0, 0]
